# Initial kernel scaffold; baseline (speedup 1.0000x reference)
import jax
import jax.numpy as jnp
from jax.experimental import pallas as pl


def kernel(nodes, atom_xyz, atom_edges, atom_edges_displacement, cell, probe_xyz, probe_edges, probe_edges_displacement, num_nodes, num_atom_edges, num_probes, num_probe_edges, params):
    raise NotImplementedError("write your pallas kernel here")


# speedup vs baseline: 1.4980x; 1.4980x over previous
import functools
import math

import jax
import jax.numpy as jnp
from jax import lax
from jax.experimental import pallas as pl
from jax.experimental.pallas import tpu as pltpu
from jax.experimental.pallas import tpu_sc as plsc

NC, NS = 2, 16
NW = NC * NS
LN2 = math.log(2.0)
CUTOFF = 4.0
STEP = 0.1
NGAUSS = 40
H = 128
CW = 48


def _mesh():
    return plsc.VectorSubcoreMesh(core_axis_name="c", subcore_axis_name="s",
                                  num_cores=NC)


def _sc_gather1(table, idx, chunk):
    M = idx.shape[0]
    C = table.shape[1]
    K = M // (NW * chunk)
    idx2 = idx.reshape(M // chunk, chunk)

    @functools.partial(
        pl.kernel,
        out_type=jax.ShapeDtypeStruct((M, C), jnp.float32),
        mesh=_mesh(),
        scratch_types=[
            pltpu.VMEM((K, chunk), jnp.int32),
            pltpu.VMEM((chunk, C), jnp.float32),
            pltpu.VMEM((chunk, C), jnp.float32),
            pltpu.SemaphoreType.DMA,
        ],
    )
    def k(t_h, i_h, o_h, iv, buf0, buf1, sem):
        w = lax.axis_index("c") * NS + lax.axis_index("s")
        pltpu.sync_copy(i_h.at[pl.ds(w * K, K)], iv)

        @pl.loop(0, K, step=2)
        def _(j):
            c0 = pltpu.async_copy(t_h.at[iv.at[j]], buf0, sem)
            c1 = pltpu.async_copy(t_h.at[iv.at[j + 1]], buf1, sem)
            base = (w * K + j) * chunk
            c0.wait()
            pltpu.sync_copy(buf0, o_h.at[pl.ds(base, chunk)])
            c1.wait()
            pltpu.sync_copy(buf1, o_h.at[pl.ds(base + chunk, chunk)])

    return k(table, idx2)


def _sc_gather2(tA, idxA, tB, idxB, chunk=128):
    M = idxA.shape[0]
    CA, CB = tA.shape[1], tB.shape[1]
    K = M // (NW * chunk)
    iA2 = idxA.reshape(M // chunk, chunk)
    iB2 = idxB.reshape(M // chunk, chunk)

    @functools.partial(
        pl.kernel,
        out_type=[jax.ShapeDtypeStruct((M, CA), jnp.float32),
                  jax.ShapeDtypeStruct((M, CB), jnp.float32)],
        mesh=_mesh(),
        scratch_types=[
            pltpu.VMEM((K, chunk), jnp.int32),
            pltpu.VMEM((K, chunk), jnp.int32),
            pltpu.VMEM((chunk, CA), jnp.float32),
            pltpu.VMEM((chunk, CB), jnp.float32),
            pltpu.VMEM((chunk, CA), jnp.float32),
            pltpu.VMEM((chunk, CB), jnp.float32),
            pltpu.SemaphoreType.DMA,
            pltpu.SemaphoreType.DMA,
        ],
    )
    def k(tA_h, iA_h, tB_h, iB_h, oA_h, oB_h, iAv, iBv,
          bufA0, bufB0, bufA1, bufB1, sA, sB):
        w = lax.axis_index("c") * NS + lax.axis_index("s")
        pltpu.sync_copy(iA_h.at[pl.ds(w * K, K)], iAv)
        pltpu.sync_copy(iB_h.at[pl.ds(w * K, K)], iBv)

        @pl.loop(0, K, step=2)
        def _(j):
            cA0 = pltpu.async_copy(tA_h.at[iAv.at[j]], bufA0, sA)
            cB0 = pltpu.async_copy(tB_h.at[iBv.at[j]], bufB0, sB)
            cA1 = pltpu.async_copy(tA_h.at[iAv.at[j + 1]], bufA1, sA)
            cB1 = pltpu.async_copy(tB_h.at[iBv.at[j + 1]], bufB1, sB)
            base = (w * K + j) * chunk
            cA0.wait()
            cB0.wait()
            pltpu.sync_copy(bufA0, oA_h.at[pl.ds(base, chunk)])
            pltpu.sync_copy(bufB0, oB_h.at[pl.ds(base, chunk)])
            cA1.wait()
            cB1.wait()
            pltpu.sync_copy(bufA1, oA_h.at[pl.ds(base + chunk, chunk)])
            pltpu.sync_copy(bufB1, oB_h.at[pl.ds(base + chunk, chunk)])

    return k(tA, iA2, tB, iB2)


def _sc_scatter_add(msg, idx, nacc, chunk=128):
    M = msg.shape[0]
    K = M // (NW * chunk)
    ZR = nacc // NS
    idx2 = idx.reshape(M // chunk, chunk)
    zeros = jnp.zeros((nacc, H), jnp.float32)

    @functools.partial(
        pl.kernel,
        out_type=[jax.ShapeDtypeStruct((nacc, H), jnp.float32),
                  jax.ShapeDtypeStruct((nacc, H), jnp.float32)],
        mesh=_mesh(),
        scratch_types=[
            pltpu.VMEM_SHARED((nacc, H), jnp.float32),
            pltpu.VMEM((K, chunk), jnp.int32),
            pltpu.VMEM((chunk, H), jnp.float32),
            pltpu.VMEM((chunk, H), jnp.float32),
            pltpu.SemaphoreType.DMA,
        ],
    )
    def k(m_h, i_h, z_h, o0_h, o1_h, accum, iv, buf0, buf1, sem):
        c = lax.axis_index("c")
        s = lax.axis_index("s")
        w = c * NS + s
        pltpu.sync_copy(i_h.at[pl.ds(w * K, K)], iv)
        pltpu.sync_copy(z_h.at[pl.ds(s * ZR, ZR)], accum.at[pl.ds(s * ZR, ZR)])
        plsc.subcore_barrier()

        @pl.loop(0, K, step=2)
        def _(j):
            base = (w * K + j) * chunk
            c0 = pltpu.async_copy(m_h.at[pl.ds(base, chunk)], buf0, sem)
            c1 = pltpu.async_copy(m_h.at[pl.ds(base + chunk, chunk)], buf1, sem)
            c0.wait()
            pltpu.sync_copy(buf0, accum.at[iv.at[j]], add=True)
            c1.wait()
            pltpu.sync_copy(buf1, accum.at[iv.at[j + 1]], add=True)

        plsc.subcore_barrier()

        @pl.when(c == 0)
        def _():
            pltpu.sync_copy(accum.at[pl.ds(s * ZR, ZR)],
                            o0_h.at[pl.ds(s * ZR, ZR)])

        @pl.when(c == 1)
        def _():
            pltpu.sync_copy(accum.at[pl.ds(s * ZR, ZR)],
                            o1_h.at[pl.ds(s * ZR, ZR)])

    return k(msg, idx2, zeros)


def _softplus(x):
    return jnp.maximum(x, 0.0) + jnp.log1p(jnp.exp(-jnp.abs(x)))


def _ssp(x):
    return _softplus(x) - LN2


def _rows(br, w):
    return pl.BlockSpec((br, w), lambda i: (i, 0))


def _full(shape):
    return pl.BlockSpec(shape, lambda i: (0, 0))


def _dot(a, b):
    return jnp.dot(a, b, preferred_element_type=jnp.float32)


def _tc_comb(psg, pdg, dispp, cellp, e_real, be=1024):
    M = psg.shape[0]

    def body(ps_r, pd_r, dp_r, cl_r, o_r):
        i = pl.program_id(0)
        disp = _dot(dp_r[...], cl_r[...])
        diff = pd_r[...] - (ps_r[...] + disp)
        dist = jnp.sqrt(jnp.sum(diff * diff, axis=1, keepdims=True) + 1e-10)
        col = lax.broadcasted_iota(jnp.int32, (be, CW), 1)
        mu = col.astype(jnp.float32) * STEP
        gauss = jnp.exp(-((dist - mu) ** 2) * (1.0 / (2.0 * STEP * STEP)))
        fcut = 0.5 * (jnp.cos(dist * (math.pi / CUTOFF)) + 1.0)
        comb = jnp.where(col < NGAUSS, gauss * fcut,
                         jnp.where(col == NGAUSS, fcut, 0.0))
        row = i * be + lax.broadcasted_iota(jnp.int32, (be, CW), 0)
        o_r[...] = jnp.where(row < e_real, comb, 0.0)

    return pl.pallas_call(
        body,
        grid=(M // be,),
        in_specs=[_rows(be, H), _rows(be, H), _rows(be, 16), _full((16, H))],
        out_specs=_rows(be, CW),
        out_shape=jax.ShapeDtypeStruct((M, CW), jnp.float32),
    )(psg, pdg, dispp, cellp)


def _tc_msg(a, b, comb, bm1, Wm2, bm2, Wfb, be=1024):
    M = a.shape[0]

    def body(a_r, b_r, c_r, b1_r, W2_r, b2_r, Wf_r, o_r):
        x1 = a_r[...] + b_r[...] + b1_r[...]
        msg = _dot(_ssp(x1), W2_r[...]) + b2_r[...]
        gf = _dot(c_r[...], Wf_r[...])
        o_r[...] = msg * gf

    return pl.pallas_call(
        body,
        grid=(M // be,),
        in_specs=[_rows(be, H), _rows(be, H), _rows(be, CW),
                  _full((1, H)), _full((H, H)), _full((1, H)), _full((CW, H))],
        out_specs=_rows(be, H),
        out_shape=jax.ShapeDtypeStruct((M, H), jnp.float32),
    )(a, b, comb, bm1, Wm2, bm2, Wfb)


def _tc_lin(x, Ws, m_out, bn=1000):
    outs = [jax.ShapeDtypeStruct((m_out, H), jnp.float32) for _ in Ws]

    def body(*refs):
        x_r = refs[0]
        w_refs = refs[1:1 + len(Ws)]
        o_refs = refs[1 + len(Ws):]
        for w_r, o_r in zip(w_refs, o_refs):
            o_r[...] = _dot(x_r[...], w_r[...])

    return pl.pallas_call(
        body,
        grid=(m_out // bn,),
        in_specs=[_rows(bn, H)] + [_full((H, H))] * len(Ws),
        out_specs=[_rows(bn, H)] * len(Ws),
        out_shape=outs,
    )(x, *Ws)


def _tc_atom_update(p0, p1, h, Ws1, bs1, Ws2, bs2, m_out, bn=1000):
    def body(p0_r, p1_r, h_r, W1_r, b1_r, W2_r, b2_r, o_r):
        ms = p0_r[...] + p1_r[...]
        t = _dot(_ssp(_dot(ms, W1_r[...]) + b1_r[...]), W2_r[...]) + b2_r[...]
        o_r[...] = h_r[...] + t

    return pl.pallas_call(
        body,
        grid=(m_out // bn,),
        in_specs=[
            _rows(bn, H), _rows(bn, H), _rows(bn, H),
            _full((H, H)), _full((1, H)), _full((H, H)), _full((1, H)),
        ],
        out_specs=_rows(bn, H),
        out_shape=jax.ShapeDtypeStruct((m_out, H), jnp.float32),
    )(p0, p1, h, Ws1, bs1, Ws2, bs2)


def _tc_probe_update(p0, p1, ps, Wg1, bg1, Wg2, bg2, Wt1, bt1, Wt2, bt2,
                     m_out, bn=1000):
    def body(p0_r, p1_r, ps_r, Wg1_r, bg1_r, Wg2_r, bg2_r,
             Wt1_r, bt1_r, Wt2_r, bt2_r, o_r):
        ms = p0_r[...] + p1_r[...]
        ga = _dot(_ssp(_dot(ps_r[...], Wg1_r[...]) + bg1_r[...]),
                  Wg2_r[...]) + bg2_r[...]
        g = 1.0 / (1.0 + jnp.exp(-ga))
        t = _dot(_ssp(_dot(ms, Wt1_r[...]) + bt1_r[...]),
                 Wt2_r[...]) + bt2_r[...]
        o_r[...] = ps_r[...] * g + (1.0 - g) * t

    return pl.pallas_call(
        body,
        grid=(m_out // bn,),
        in_specs=[
            _rows(bn, H), _rows(bn, H), _rows(bn, H),
            _full((H, H)), _full((1, H)), _full((H, H)), _full((1, H)),
            _full((H, H)), _full((1, H)), _full((H, H)), _full((1, H)),
        ],
        out_specs=_rows(bn, H),
        out_shape=jax.ShapeDtypeStruct((m_out, H), jnp.float32),
    )(p0, p1, ps, Wg1, bg1, Wg2, bg2, Wt1, bt1, Wt2, bt2)


def _tc_readout(ps, Wr1, br1, Wr2p, br2p, m_out, bn=1000):
    def body(ps_r, W1_r, b1_r, W2_r, b2_r, o_r):
        o_r[...] = _dot(_ssp(_dot(ps_r[...], W1_r[...]) + b1_r[...]),
                        W2_r[...]) + b2_r[...]

    return pl.pallas_call(
        body,
        grid=(m_out // bn,),
        in_specs=[_rows(bn, H), _full((H, H)), _full((1, H)),
                  _full((H, H)), _full((1, H))],
        out_specs=_rows(bn, H),
        out_shape=jax.ShapeDtypeStruct((m_out, H), jnp.float32),
    )(ps, Wr1, br1, Wr2p, br2p)


def _pad1(x, m, fill):
    return jnp.concatenate(
        [x, jnp.full((m - x.shape[0],), fill, x.dtype)]) if m > x.shape[0] else x


def kernel(nodes, atom_xyz, atom_edges, atom_edges_displacement, cell,
           probe_xyz, probe_edges, probe_edges_displacement,
           num_nodes, num_atom_edges, num_probes, num_probe_edges, params):
    N = atom_xyz.shape[1]
    E = atom_edges.shape[1]
    P = probe_xyz.shape[1]
    PE = probe_edges.shape[1]

    nodes1 = nodes[0].astype(jnp.int32)
    asrc = atom_edges[0, :, 0].astype(jnp.int32)
    adst = atom_edges[0, :, 1].astype(jnp.int32)
    psrc = probe_edges[0, :, 0].astype(jnp.int32)
    pdst = probe_edges[0, :, 1].astype(jnp.int32)

    EG = NW * 128
    EP = ((E + EG - 1) // EG) * EG
    PEP = ((PE + EG - 1) // EG) * EG
    NGC = 40
    NGP = ((N + NW * 8 * NGC - 1) // (NW * 8 * NGC)) * (NW * 8 * NGC)
    BN = 1000
    NACC = ((max(N, P) + 1 + 127) // 128) * 128

    asrc_p = _pad1(asrc, EP, 0)
    adst_p = _pad1(adst, EP, 0)
    adst_s = _pad1(adst, EP, N)
    psrc_p = _pad1(psrc, PEP, 0)
    pdst_p = _pad1(pdst, PEP, 0)
    pdst_s = _pad1(pdst, PEP, P)
    nodes_p = _pad1(nodes1, NGP, 0)

    h = _sc_gather1(params['embedding'], nodes_p, NGC)

    ptab_a = jnp.pad(atom_xyz[0], ((0, 0), (0, H - 3)))
    ptab_p = jnp.pad(probe_xyz[0], ((0, 0), (0, H - 3)))
    disp_a = jnp.pad(atom_edges_displacement[0],
                     ((0, EP - E), (0, 13)))
    disp_p = jnp.pad(probe_edges_displacement[0],
                     ((0, PEP - PE), (0, 13)))
    cellp = jnp.pad(cell[0], ((0, 13), (0, H - 3)))

    psA, pdA = _sc_gather2(ptab_a, asrc_p, ptab_a, adst_p)
    comb_a = _tc_comb(psA, pdA, disp_a, cellp, E)
    psP, pdP = _sc_gather2(ptab_a, psrc_p, ptab_p, pdst_p)
    comb_p = _tc_comb(psP, pdP, disp_p, cellp, PE)

    def prep(p):
        Wa = p['Wm1'][:H]
        Wb = p['Wm1'][H:]
        Wfb = jnp.pad(jnp.concatenate([p['Wf'], p['bf'][None]], axis=0),
                      ((0, CW - NGAUSS - 1), (0, 0)))
        return Wa, Wb, Wfb

    ap = [prep(p) for p in params['atom']]
    pp = [prep(q) for q in params['probe']]

    T = len(params['atom'])
    A, B = _tc_lin(h, [ap[0][0], ap[0][1]], N)
    reps_A = []
    for i in range(T):
        p = params['atom'][i]
        a_s, b_d = _sc_gather2(A, asrc_p, B, adst_p)
        msg = _tc_msg(a_s, b_d, comb_a, p['bm1'][None], p['Wm2'],
                      p['bm2'][None], ap[i][2])
        sc0, sc1 = _sc_scatter_add(msg, adst_s, NACC)
        h = _tc_atom_update(sc0, sc1, h, p['Ws1'], p['bs1'][None],
                            p['Ws2'], p['bs2'][None], N, BN)
        reps_A.append(_tc_lin(h, [pp[i][0]], N)[0])
        if i + 1 < T:
            A, B = _tc_lin(h, [ap[i + 1][0], ap[i + 1][1]], N)

    ps = jnp.zeros((P, H), jnp.float32)
    Bp = jnp.zeros((P, H), jnp.float32)
    for i in range(T):
        q = params['probe'][i]
        a_s, b_d = _sc_gather2(reps_A[i], psrc_p, Bp, pdst_p)
        msg = _tc_msg(a_s, b_d, comb_p, q['bm1'][None], q['Wm2'],
                      q['bm2'][None], pp[i][2])
        sc0, sc1 = _sc_scatter_add(msg, pdst_s, NACC)
        ps = _tc_probe_update(sc0, sc1, ps, q['Wg1'], q['bg1'][None],
                              q['Wg2'], q['bg2'][None], q['Wt1'],
                              q['bt1'][None], q['Wt2'], q['bt2'][None],
                              P, BN)
        if i + 1 < T:
            Bp = _tc_lin(ps, [pp[i + 1][1]], P)[0]

    ro = params['readout']
    Wr2p = jnp.pad(ro['Wr2'], ((0, 0), (0, H - ro['Wr2'].shape[1])))
    br2p = jnp.pad(ro['br2'][None], ((0, 0), (0, H - ro['br2'].shape[0])))
    out = _tc_readout(ps, ro['Wr1'], ro['br1'][None], Wr2p, br2p, P)
    return out[:, 0].reshape(1, P)
